# Initial kernel scaffold; baseline (speedup 1.0000x reference)
import jax
import jax.numpy as jnp
from jax.experimental import pallas as pl


def kernel(x, g_fc1_w, g_fc1_b, g_bn1_g, g_bn1_b, g_gc_w, g_gc_b, g_bn2_g, g_bn2_b, g_fc2_w, g_fc2_b, g_bn3_g, g_bn3_b, f_fc1_w, f_fc1_b, f_bn1_g, f_bn1_b, f_fc2_w, f_fc2_b, f_bn2_g, f_bn2_b):
    raise NotImplementedError("write your pallas kernel here")


# speedup vs baseline: 294.0031x; 294.0031x over previous
import functools

import jax
import jax.numpy as jnp
from jax import lax
from jax.experimental import pallas as pl
from jax.experimental.pallas import tpu as pltpu
from jax.experimental.pallas import tpu_sc as plsc

BB, CC, HH, WW = 8, 96, 32, 32
NN = HH * WW
NT = BB * NN
KK = 9
HG = 2 * CC
HF = 4 * CC
EPS_BN = 1e-5
_HI = None

_RC = 2048
_G = NT // _RC

_NWORK = 32
_NPW = NT // _NWORK
_CH = 8
_NCH = _NPW // _CH
_ROWS = _CH * KK
_HGP = 256


def _gelu(x):
    return 0.5 * x * (1.0 + lax.erf(x * (2.0 ** -0.5)))


def _sums(z):
    return jnp.stack([jnp.sum(z, axis=0), jnp.sum(z * z, axis=0)])


def _acc(i, s_ref, part):
    @pl.when(i == 0)
    def _():
        s_ref[...] = part

    @pl.when(i != 0)
    def _():
        s_ref[...] += part


def _bn_from(z, s, g, b):
    mean = s[0:1] / NT
    var = s[1:2] / NT - mean * mean
    return g * ((z - mean) * lax.rsqrt(var + EPS_BN)) + b


def _mm(x, w):
    return jnp.dot(x, w, precision=_HI, preferred_element_type=jnp.float32)


def _p1_body(xt_ref, w1t_ref, b1_ref, z1_ref, s1_ref):
    i = pl.program_id(0)
    z = _mm(xt_ref[...], w1t_ref[...]) + b1_ref[...]
    z1_ref[...] = z
    _acc(i, s1_ref, _sums(z))


def _p2_body(z1_ref, s1_ref, g1_ref, be1_ref, wa_ref, wb_ref, bg_ref,
             feat_ref, a_ref, bmp_ref):
    y1 = _bn_from(z1_ref[...], s1_ref[...], g1_ref[...], be1_ref[...])
    nrm = jnp.sqrt(jnp.sum(y1 * y1, axis=1, keepdims=True))
    feat_ref[...] = y1 / jnp.maximum(nrm, 1e-12)
    a_ref[...] = _mm(y1, wa_ref[...]) + bg_ref[...]
    bmp_ref[...] = jnp.concatenate(
        [_mm(y1, wb_ref[...]), jnp.zeros((_RC, _HGP - HG), jnp.float32)],
        axis=1)


def _knn_body(feat_ref, idx_ref):
    b = pl.program_id(0)
    f = feat_ref[0]
    sq = jnp.sum(f * f, axis=1, keepdims=True)
    prod = lax.dot_general(f, f, (((1,), (1,)), ((), ())), precision=_HI,
                           preferred_element_type=jnp.float32)
    d = sq - 2.0 * prod + jnp.reshape(sq, (1, NN))
    iota = lax.broadcasted_iota(jnp.int32, (NN, NN), 1)
    cols = []
    for _ in range(KK):
        m = jnp.min(d, axis=1, keepdims=True)
        idx = jnp.min(jnp.where(d == m, iota, NN), axis=1)
        cols.append(idx + b * NN)
        d = jnp.where(iota == idx[:, None], jnp.inf, d)
    idx_ref[0] = jnp.stack(cols, axis=1)


def _sc_body(idx_hbm, bm_hbm, out_hbm, idx_v, rows_v, out_v, sem):
    wid = lax.axis_index("s") * 2 + lax.axis_index("c")
    node_base = wid * _NPW

    def chunk(ci, carry):
        nb = node_base + ci * _CH
        pltpu.sync_copy(idx_hbm.at[pl.ds(nb * KK, _ROWS)], idx_v)
        pltpu.async_copy(bm_hbm.at[idx_v], rows_v, sem).wait()
        for n in range(_CH):
            for dp in range(HG // 16):
                sl = pl.ds(dp * 16, 16)
                acc = rows_v[n * KK, sl]
                for j in range(1, KK):
                    acc = jnp.maximum(acc, rows_v[n * KK + j, sl])
                out_v[n, sl] = acc
        pltpu.sync_copy(out_v, out_hbm.at[pl.ds(nb, _CH)])
        return carry

    lax.fori_loop(0, _NCH, chunk, 0)


@functools.cache
def _sc_gather_max():
    mesh = plsc.VectorSubcoreMesh(core_axis_name="c", subcore_axis_name="s")
    return pl.kernel(
        _sc_body,
        mesh=mesh,
        out_type=jax.ShapeDtypeStruct((NT, HG), jnp.float32),
        scratch_types=[
            pltpu.VMEM((_ROWS,), jnp.int32),
            pltpu.VMEM((_ROWS, _HGP), jnp.float32),
            pltpu.VMEM((_CH, HG), jnp.float32),
            pltpu.SemaphoreType.DMA,
        ],
    )


def _p3_body(a_ref, gm_ref, s2_ref):
    i = pl.program_id(0)
    _acc(i, s2_ref, _sums(a_ref[...] + gm_ref[...]))


def _p4_body(a_ref, gm_ref, s2_ref, g2_ref, b2_ref, w2t_ref, bc2_ref,
             z2_ref, s3_ref):
    i = pl.program_id(0)
    h = _gelu(_bn_from(a_ref[...] + gm_ref[...], s2_ref[...], g2_ref[...],
                       b2_ref[...]))
    z = _mm(h, w2t_ref[...]) + bc2_ref[...]
    z2_ref[...] = z
    _acc(i, s3_ref, _sums(z))


def _p5_body(z2_ref, s3_ref, g3_ref, b3_ref, xt_ref, wf1t_ref, bf1_ref,
             x2_ref, u_ref, s4_ref):
    i = pl.program_id(0)
    x2 = _bn_from(z2_ref[...], s3_ref[...], g3_ref[...], b3_ref[...]) \
        + xt_ref[...]
    x2_ref[...] = x2
    u = _mm(x2, wf1t_ref[...]) + bf1_ref[...]
    u_ref[...] = u
    _acc(i, s4_ref, _sums(u))


def _p6_body(u_ref, s4_ref, gf1_ref, bef1_ref, wf2t_ref, bf2_ref,
             v_ref, s5_ref):
    i = pl.program_id(0)
    hu = _gelu(_bn_from(u_ref[...], s4_ref[...], gf1_ref[...], bef1_ref[...]))
    v = _mm(hu, wf2t_ref[...]) + bf2_ref[...]
    v_ref[...] = v
    _acc(i, s5_ref, _sums(v))


def _p7_body(v_ref, s5_ref, gf2_ref, bef2_ref, x2_ref, out_ref):
    out_ref[...] = _bn_from(v_ref[...], s5_ref[...], gf2_ref[...],
                            bef2_ref[...]) + x2_ref[...]


def _row(v):
    return v.reshape(1, -1)


def _chunk_spec(width):
    return pl.BlockSpec((_RC, width), lambda i: (i, 0))


def _full_spec(shape):
    return pl.BlockSpec(shape, lambda i: (0, 0))


def _sum_spec(width):
    return pl.BlockSpec((2, width), lambda i: (0, 0))


def kernel(x, g_fc1_w, g_fc1_b, g_bn1_g, g_bn1_b, g_gc_w, g_gc_b, g_bn2_g,
           g_bn2_b, g_fc2_w, g_fc2_b, g_bn3_g, g_bn3_b, f_fc1_w, f_fc1_b,
           f_bn1_g, f_bn1_b, f_fc2_w, f_fc2_b, f_bn2_g, f_bn2_b):
    xt = x.reshape(BB, CC, NN).transpose(0, 2, 1).reshape(NT, CC)
    wa = (g_gc_w[:, :CC] - g_gc_w[:, CC:]).T
    wb = g_gc_w[:, CC:].T
    f32 = jnp.float32

    z1, s1 = pl.pallas_call(
        _p1_body, grid=(_G,),
        in_specs=[_chunk_spec(CC), _full_spec((CC, CC)), _full_spec((1, CC))],
        out_specs=[_chunk_spec(CC), _sum_spec(CC)],
        out_shape=[jax.ShapeDtypeStruct((NT, CC), f32),
                   jax.ShapeDtypeStruct((2, CC), f32)],
    )(xt, g_fc1_w.T, _row(g_fc1_b))

    feat, a, bmp = pl.pallas_call(
        _p2_body, grid=(_G,),
        in_specs=[_chunk_spec(CC), _sum_spec(CC), _full_spec((1, CC)),
                  _full_spec((1, CC)), _full_spec((CC, HG)),
                  _full_spec((CC, HG)), _full_spec((1, HG))],
        out_specs=[_chunk_spec(CC), _chunk_spec(HG), _chunk_spec(_HGP)],
        out_shape=[jax.ShapeDtypeStruct((NT, CC), f32),
                   jax.ShapeDtypeStruct((NT, HG), f32),
                   jax.ShapeDtypeStruct((NT, _HGP), f32)],
    )(z1, s1, _row(g_bn1_g), _row(g_bn1_b), wa, wb, _row(g_gc_b))

    nn_idx = pl.pallas_call(
        _knn_body, grid=(BB,),
        in_specs=[pl.BlockSpec((1, NN, CC), lambda b: (b, 0, 0))],
        out_specs=pl.BlockSpec((1, NN, KK), lambda b: (b, 0, 0)),
        out_shape=jax.ShapeDtypeStruct((BB, NN, KK), jnp.int32),
    )(feat.reshape(BB, NN, CC))

    gm = _sc_gather_max()(nn_idx.reshape(NT * KK), bmp)

    s2 = pl.pallas_call(
        _p3_body, grid=(_G,),
        in_specs=[_chunk_spec(HG), _chunk_spec(HG)],
        out_specs=_sum_spec(HG),
        out_shape=jax.ShapeDtypeStruct((2, HG), f32),
    )(a, gm)

    z2, s3 = pl.pallas_call(
        _p4_body, grid=(_G,),
        in_specs=[_chunk_spec(HG), _chunk_spec(HG), _sum_spec(HG),
                  _full_spec((1, HG)), _full_spec((1, HG)),
                  _full_spec((HG, CC)), _full_spec((1, CC))],
        out_specs=[_chunk_spec(CC), _sum_spec(CC)],
        out_shape=[jax.ShapeDtypeStruct((NT, CC), f32),
                   jax.ShapeDtypeStruct((2, CC), f32)],
    )(a, gm, s2, _row(g_bn2_g), _row(g_bn2_b), g_fc2_w.T, _row(g_fc2_b))

    x2, u, s4 = pl.pallas_call(
        _p5_body, grid=(_G,),
        in_specs=[_chunk_spec(CC), _sum_spec(CC), _full_spec((1, CC)),
                  _full_spec((1, CC)), _chunk_spec(CC),
                  _full_spec((CC, HF)), _full_spec((1, HF))],
        out_specs=[_chunk_spec(CC), _chunk_spec(HF), _sum_spec(HF)],
        out_shape=[jax.ShapeDtypeStruct((NT, CC), f32),
                   jax.ShapeDtypeStruct((NT, HF), f32),
                   jax.ShapeDtypeStruct((2, HF), f32)],
    )(z2, s3, _row(g_bn3_g), _row(g_bn3_b), xt, f_fc1_w.T, _row(f_fc1_b))

    v, s5 = pl.pallas_call(
        _p6_body, grid=(_G,),
        in_specs=[_chunk_spec(HF), _sum_spec(HF), _full_spec((1, HF)),
                  _full_spec((1, HF)), _full_spec((HF, CC)),
                  _full_spec((1, CC))],
        out_specs=[_chunk_spec(CC), _sum_spec(CC)],
        out_shape=[jax.ShapeDtypeStruct((NT, CC), f32),
                   jax.ShapeDtypeStruct((2, CC), f32)],
    )(u, s4, _row(f_bn1_g), _row(f_bn1_b), f_fc2_w.T, _row(f_fc2_b))

    out = pl.pallas_call(
        _p7_body, grid=(_G,),
        in_specs=[_chunk_spec(CC), _sum_spec(CC), _full_spec((1, CC)),
                  _full_spec((1, CC)), _chunk_spec(CC)],
        out_specs=_chunk_spec(CC),
        out_shape=jax.ShapeDtypeStruct((NT, CC), f32),
    )(v, s5, _row(f_bn2_g), _row(f_bn2_b), x2)

    return out.reshape(BB, NN, CC).transpose(0, 2, 1).reshape(BB, CC, NN, 1)


# speedup vs baseline: 332.1141x; 1.1296x over previous
import functools

import jax
import jax.numpy as jnp
from jax import lax
from jax.experimental import pallas as pl
from jax.experimental.pallas import tpu as pltpu
from jax.experimental.pallas import tpu_sc as plsc

BB, CC, HH, WW = 8, 96, 32, 32
NN = HH * WW
NT = BB * NN
KK = 9
HG = 2 * CC
HF = 4 * CC
EPS_BN = 1e-5
_HI = None

_RC = 2048
_G = NT // _RC

_NWORK = 32
_NPW = NT // _NWORK
_CH = 8
_NCH = _NPW // _CH
_ROWS = _CH * KK
_HGP = 256


def _gelu(x):
    return 0.5 * x * (1.0 + lax.erf(x * (2.0 ** -0.5)))


def _sums(z):
    return jnp.stack([jnp.sum(z, axis=0), jnp.sum(z * z, axis=0)])


def _acc(i, s_ref, part):
    @pl.when(i == 0)
    def _():
        s_ref[...] = part

    @pl.when(i != 0)
    def _():
        s_ref[...] += part


def _bn_from(z, s, g, b):
    mean = s[0:1] / NT
    var = s[1:2] / NT - mean * mean
    return g * ((z - mean) * lax.rsqrt(var + EPS_BN)) + b


def _mm(x, w):
    return jnp.dot(x, w, precision=_HI, preferred_element_type=jnp.float32)


def _p1_body(xt_ref, w1t_ref, b1_ref, z1_ref, s1_ref):
    i = pl.program_id(0)
    z = _mm(xt_ref[...], w1t_ref[...]) + b1_ref[...]
    z1_ref[...] = z
    _acc(i, s1_ref, _sums(z))


def _p2_body(z1_ref, s1_ref, g1_ref, be1_ref, wa_ref, wb_ref, bg_ref,
             feat_ref, a_ref, bmp_ref):
    y1 = _bn_from(z1_ref[...], s1_ref[...], g1_ref[...], be1_ref[...])
    nrm = jnp.sqrt(jnp.sum(y1 * y1, axis=1, keepdims=True))
    feat_ref[...] = y1 / jnp.maximum(nrm, 1e-12)
    a_ref[...] = _mm(y1, wa_ref[...]) + bg_ref[...]
    bmp_ref[...] = jnp.concatenate(
        [_mm(y1, wb_ref[...]), jnp.zeros((_RC, _HGP - HG), jnp.float32)],
        axis=1)


def _knn_body(feat_ref, idx_ref):
    b = pl.program_id(0)
    f = feat_ref[0]
    sq = jnp.sum(f * f, axis=1, keepdims=True)
    prod = lax.dot_general(f, f, (((1,), (1,)), ((), ())), precision=_HI,
                           preferred_element_type=jnp.float32)
    d = sq - 2.0 * prod + jnp.reshape(sq, (1, NN))
    iota = lax.broadcasted_iota(jnp.int32, (NN, NN), 1)
    cols = []
    for _ in range(KK):
        m = jnp.min(d, axis=1, keepdims=True)
        idx = jnp.min(jnp.where(d == m, iota, NN), axis=1)
        cols.append(idx + b * NN)
        d = jnp.where(iota == idx[:, None], jnp.inf, d)
    idx_ref[0] = jnp.stack(cols, axis=1)


def _sc_body(idx_hbm, bm_hbm, out_hbm, idx0, idx1, rows0, rows1, out_v,
             sem0, sem1):
    wid = lax.axis_index("s") * 2 + lax.axis_index("c")
    node_base = wid * _NPW

    def start(c, idx_v, rows_v, sem):
        nb = node_base + c * _CH
        pltpu.sync_copy(idx_hbm.at[pl.ds(nb * KK, _ROWS)], idx_v)
        pltpu.async_copy(bm_hbm.at[idx_v], rows_v, sem)

    def wait(idx_v, rows_v, sem):
        pltpu.make_async_copy(bm_hbm.at[idx_v], rows_v, sem).wait()

    def combine(c, rows_v):
        nb = node_base + c * _CH
        for n in range(_CH):
            for dp in range(HG // 16):
                sl = pl.ds(dp * 16, 16)
                acc = rows_v[n * KK, sl]
                for j in range(1, KK):
                    acc = jnp.maximum(acc, rows_v[n * KK + j, sl])
                out_v[n, sl] = acc
        pltpu.sync_copy(out_v, out_hbm.at[pl.ds(nb, _CH)])

    start(0, idx0, rows0, sem0)

    def pair(i, carry):
        c0 = 2 * i
        wait(idx0, rows0, sem0)
        start(c0 + 1, idx1, rows1, sem1)
        combine(c0, rows0)
        wait(idx1, rows1, sem1)

        @pl.when(i + 1 < _NCH // 2)
        def _():
            start(c0 + 2, idx0, rows0, sem0)

        combine(c0 + 1, rows1)
        return carry

    lax.fori_loop(0, _NCH // 2, pair, 0)


@functools.cache
def _sc_gather_max():
    mesh = plsc.VectorSubcoreMesh(core_axis_name="c", subcore_axis_name="s")
    return pl.kernel(
        _sc_body,
        mesh=mesh,
        out_type=jax.ShapeDtypeStruct((NT, HG), jnp.float32),
        scratch_types=[
            pltpu.VMEM((_ROWS,), jnp.int32),
            pltpu.VMEM((_ROWS,), jnp.int32),
            pltpu.VMEM((_ROWS, _HGP), jnp.float32),
            pltpu.VMEM((_ROWS, _HGP), jnp.float32),
            pltpu.VMEM((_CH, HG), jnp.float32),
            pltpu.SemaphoreType.DMA,
            pltpu.SemaphoreType.DMA,
        ],
    )


def _p3_body(a_ref, gm_ref, s2_ref):
    i = pl.program_id(0)
    _acc(i, s2_ref, _sums(a_ref[...] + gm_ref[...]))


def _p4_body(a_ref, gm_ref, s2_ref, g2_ref, b2_ref, w2t_ref, bc2_ref,
             z2_ref, s3_ref):
    i = pl.program_id(0)
    h = _gelu(_bn_from(a_ref[...] + gm_ref[...], s2_ref[...], g2_ref[...],
                       b2_ref[...]))
    z = _mm(h, w2t_ref[...]) + bc2_ref[...]
    z2_ref[...] = z
    _acc(i, s3_ref, _sums(z))


def _p5_body(z2_ref, s3_ref, g3_ref, b3_ref, xt_ref, wf1t_ref, bf1_ref,
             x2_ref, u_ref, s4_ref):
    i = pl.program_id(0)
    x2 = _bn_from(z2_ref[...], s3_ref[...], g3_ref[...], b3_ref[...]) \
        + xt_ref[...]
    x2_ref[...] = x2
    u = _mm(x2, wf1t_ref[...]) + bf1_ref[...]
    u_ref[...] = u
    _acc(i, s4_ref, _sums(u))


def _p6_body(u_ref, s4_ref, gf1_ref, bef1_ref, wf2t_ref, bf2_ref,
             v_ref, s5_ref):
    i = pl.program_id(0)
    hu = _gelu(_bn_from(u_ref[...], s4_ref[...], gf1_ref[...], bef1_ref[...]))
    v = _mm(hu, wf2t_ref[...]) + bf2_ref[...]
    v_ref[...] = v
    _acc(i, s5_ref, _sums(v))


def _p7_body(v_ref, s5_ref, gf2_ref, bef2_ref, x2_ref, out_ref):
    out_ref[...] = _bn_from(v_ref[...], s5_ref[...], gf2_ref[...],
                            bef2_ref[...]) + x2_ref[...]


def _row(v):
    return v.reshape(1, -1)


def _chunk_spec(width):
    return pl.BlockSpec((_RC, width), lambda i: (i, 0))


def _full_spec(shape):
    return pl.BlockSpec(shape, lambda i: (0, 0))


def _sum_spec(width):
    return pl.BlockSpec((2, width), lambda i: (0, 0))


def kernel(x, g_fc1_w, g_fc1_b, g_bn1_g, g_bn1_b, g_gc_w, g_gc_b, g_bn2_g,
           g_bn2_b, g_fc2_w, g_fc2_b, g_bn3_g, g_bn3_b, f_fc1_w, f_fc1_b,
           f_bn1_g, f_bn1_b, f_fc2_w, f_fc2_b, f_bn2_g, f_bn2_b):
    xt = x.reshape(BB, CC, NN).transpose(0, 2, 1).reshape(NT, CC)
    wa = (g_gc_w[:, :CC] - g_gc_w[:, CC:]).T
    wb = g_gc_w[:, CC:].T
    f32 = jnp.float32

    z1, s1 = pl.pallas_call(
        _p1_body, grid=(_G,),
        in_specs=[_chunk_spec(CC), _full_spec((CC, CC)), _full_spec((1, CC))],
        out_specs=[_chunk_spec(CC), _sum_spec(CC)],
        out_shape=[jax.ShapeDtypeStruct((NT, CC), f32),
                   jax.ShapeDtypeStruct((2, CC), f32)],
    )(xt, g_fc1_w.T, _row(g_fc1_b))

    feat, a, bmp = pl.pallas_call(
        _p2_body, grid=(_G,),
        in_specs=[_chunk_spec(CC), _sum_spec(CC), _full_spec((1, CC)),
                  _full_spec((1, CC)), _full_spec((CC, HG)),
                  _full_spec((CC, HG)), _full_spec((1, HG))],
        out_specs=[_chunk_spec(CC), _chunk_spec(HG), _chunk_spec(_HGP)],
        out_shape=[jax.ShapeDtypeStruct((NT, CC), f32),
                   jax.ShapeDtypeStruct((NT, HG), f32),
                   jax.ShapeDtypeStruct((NT, _HGP), f32)],
    )(z1, s1, _row(g_bn1_g), _row(g_bn1_b), wa, wb, _row(g_gc_b))

    nn_idx = pl.pallas_call(
        _knn_body, grid=(BB,),
        in_specs=[pl.BlockSpec((1, NN, CC), lambda b: (b, 0, 0))],
        out_specs=pl.BlockSpec((1, NN, KK), lambda b: (b, 0, 0)),
        out_shape=jax.ShapeDtypeStruct((BB, NN, KK), jnp.int32),
    )(feat.reshape(BB, NN, CC))

    gm = _sc_gather_max()(nn_idx.reshape(NT * KK), bmp)

    s2 = pl.pallas_call(
        _p3_body, grid=(_G,),
        in_specs=[_chunk_spec(HG), _chunk_spec(HG)],
        out_specs=_sum_spec(HG),
        out_shape=jax.ShapeDtypeStruct((2, HG), f32),
    )(a, gm)

    z2, s3 = pl.pallas_call(
        _p4_body, grid=(_G,),
        in_specs=[_chunk_spec(HG), _chunk_spec(HG), _sum_spec(HG),
                  _full_spec((1, HG)), _full_spec((1, HG)),
                  _full_spec((HG, CC)), _full_spec((1, CC))],
        out_specs=[_chunk_spec(CC), _sum_spec(CC)],
        out_shape=[jax.ShapeDtypeStruct((NT, CC), f32),
                   jax.ShapeDtypeStruct((2, CC), f32)],
    )(a, gm, s2, _row(g_bn2_g), _row(g_bn2_b), g_fc2_w.T, _row(g_fc2_b))

    x2, u, s4 = pl.pallas_call(
        _p5_body, grid=(_G,),
        in_specs=[_chunk_spec(CC), _sum_spec(CC), _full_spec((1, CC)),
                  _full_spec((1, CC)), _chunk_spec(CC),
                  _full_spec((CC, HF)), _full_spec((1, HF))],
        out_specs=[_chunk_spec(CC), _chunk_spec(HF), _sum_spec(HF)],
        out_shape=[jax.ShapeDtypeStruct((NT, CC), f32),
                   jax.ShapeDtypeStruct((NT, HF), f32),
                   jax.ShapeDtypeStruct((2, HF), f32)],
    )(z2, s3, _row(g_bn3_g), _row(g_bn3_b), xt, f_fc1_w.T, _row(f_fc1_b))

    v, s5 = pl.pallas_call(
        _p6_body, grid=(_G,),
        in_specs=[_chunk_spec(HF), _sum_spec(HF), _full_spec((1, HF)),
                  _full_spec((1, HF)), _full_spec((HF, CC)),
                  _full_spec((1, CC))],
        out_specs=[_chunk_spec(CC), _sum_spec(CC)],
        out_shape=[jax.ShapeDtypeStruct((NT, CC), f32),
                   jax.ShapeDtypeStruct((2, CC), f32)],
    )(u, s4, _row(f_bn1_g), _row(f_bn1_b), f_fc2_w.T, _row(f_fc2_b))

    out = pl.pallas_call(
        _p7_body, grid=(_G,),
        in_specs=[_chunk_spec(CC), _sum_spec(CC), _full_spec((1, CC)),
                  _full_spec((1, CC)), _chunk_spec(CC)],
        out_specs=_chunk_spec(CC),
        out_shape=jax.ShapeDtypeStruct((NT, CC), f32),
    )(v, s5, _row(f_bn2_g), _row(f_bn2_b), x2)

    return out.reshape(BB, NN, CC).transpose(0, 2, 1).reshape(BB, CC, NN, 1)


# speedup vs baseline: 355.6984x; 1.0710x over previous
import functools

import jax
import jax.numpy as jnp
from jax import lax
from jax.experimental import pallas as pl
from jax.experimental.pallas import tpu as pltpu
from jax.experimental.pallas import tpu_sc as plsc

BB, CC, HH, WW = 8, 96, 32, 32
NN = HH * WW
NT = BB * NN
KK = 9
HG = 2 * CC
HF = 4 * CC
EPS_BN = 1e-5
_HI = None

_RC = 2048
_G = NT // _RC

_NWORK = 32
_NPW = NT // _NWORK
_CH = 8
_NCH = _NPW // _CH
_ROWS = _CH * KK
_HGP = 256


def _gelu(x):
    return 0.5 * x * (1.0 + lax.erf(x * (2.0 ** -0.5)))


def _sums(z):
    return jnp.stack([jnp.sum(z, axis=0), jnp.sum(z * z, axis=0)])


def _acc(i, s_ref, part):
    @pl.when(i == 0)
    def _():
        s_ref[...] = part

    @pl.when(i != 0)
    def _():
        s_ref[...] += part


def _bn_from(z, s, g, b):
    mean = s[0:1] / NT
    var = s[1:2] / NT - mean * mean
    return g * ((z - mean) * lax.rsqrt(var + EPS_BN)) + b


def _mm(x, w):
    return jnp.dot(x, w, precision=_HI, preferred_element_type=jnp.float32)


def _p1_body(xt_ref, w1t_ref, b1_ref, z1_ref, s1_ref):
    i = pl.program_id(0)
    z = _mm(xt_ref[...], w1t_ref[...]) + b1_ref[...]
    z1_ref[...] = z
    _acc(i, s1_ref, _sums(z))


def _p2_body(z1_ref, s1_ref, g1_ref, be1_ref, wa_ref, wb_ref, bg_ref,
             feat_ref, a_ref, bmp_ref):
    y1 = _bn_from(z1_ref[...], s1_ref[...], g1_ref[...], be1_ref[...])
    nrm = jnp.sqrt(jnp.sum(y1 * y1, axis=1, keepdims=True))
    feat_ref[...] = y1 / jnp.maximum(nrm, 1e-12)
    a_ref[...] = _mm(y1, wa_ref[...]) + bg_ref[...]
    bmp_ref[...] = jnp.concatenate(
        [_mm(y1, wb_ref[...]), jnp.zeros((_RC, _HGP - HG), jnp.float32)],
        axis=1)


def _knn_body(feat_ref, idx_ref):
    b = pl.program_id(0)
    f = feat_ref[0]
    sq = jnp.sum(f * f, axis=1, keepdims=True)
    prod = lax.dot_general(f, f, (((1,), (1,)), ((), ())), precision=_HI,
                           preferred_element_type=jnp.float32)
    d = sq - 2.0 * prod + jnp.reshape(sq, (1, NN))
    iota = lax.broadcasted_iota(jnp.int32, (NN, NN), 1)
    cols = []
    for _ in range(KK):
        m = jnp.min(d, axis=1, keepdims=True)
        idx = jnp.min(jnp.where(d == m, iota, NN), axis=1)
        cols.append(idx + b * NN)
        d = jnp.where(iota == idx[:, None], jnp.inf, d)
    idx_ref[0] = jnp.stack(cols, axis=1)


def _sc_body(idx_hbm, bm_hbm, out_hbm, idx_all, rows0, rows1, out_v,
             sem0, sem1):
    wid = lax.axis_index("s") * 2 + lax.axis_index("c")
    node_base = wid * _NPW
    pltpu.sync_copy(idx_hbm.at[pl.ds(node_base * KK, _NPW * KK)], idx_all)

    def start(c, rows_v, sem):
        pltpu.async_copy(bm_hbm.at[idx_all.at[pl.ds(c * _ROWS, _ROWS)]],
                         rows_v, sem)

    def wait(c, rows_v, sem):
        pltpu.make_async_copy(bm_hbm.at[idx_all.at[pl.ds(c * _ROWS, _ROWS)]],
                              rows_v, sem).wait()

    def combine(c, rows_v):
        nb = node_base + c * _CH
        for n in range(_CH):
            for dp in range(HG // 16):
                sl = pl.ds(dp * 16, 16)
                acc = rows_v[n * KK, sl]
                for j in range(1, KK):
                    acc = jnp.maximum(acc, rows_v[n * KK + j, sl])
                out_v[n, sl] = acc
        pltpu.sync_copy(out_v, out_hbm.at[pl.ds(nb, _CH)])

    start(0, rows0, sem0)

    def pair(i, carry):
        c0 = 2 * i
        wait(c0, rows0, sem0)
        start(c0 + 1, rows1, sem1)
        combine(c0, rows0)
        wait(c0 + 1, rows1, sem1)

        @pl.when(i + 1 < _NCH // 2)
        def _():
            start(c0 + 2, rows0, sem0)

        combine(c0 + 1, rows1)
        return carry

    lax.fori_loop(0, _NCH // 2, pair, 0)


@functools.cache
def _sc_gather_max():
    mesh = plsc.VectorSubcoreMesh(core_axis_name="c", subcore_axis_name="s")
    return pl.kernel(
        _sc_body,
        mesh=mesh,
        out_type=jax.ShapeDtypeStruct((NT, HG), jnp.float32),
        scratch_types=[
            pltpu.VMEM((_NPW * KK,), jnp.int32),
            pltpu.VMEM((_ROWS, _HGP), jnp.float32),
            pltpu.VMEM((_ROWS, _HGP), jnp.float32),
            pltpu.VMEM((_CH, HG), jnp.float32),
            pltpu.SemaphoreType.DMA,
            pltpu.SemaphoreType.DMA,
        ],
    )


def _p3_body(a_ref, gm_ref, s2_ref):
    i = pl.program_id(0)
    _acc(i, s2_ref, _sums(a_ref[...] + gm_ref[...]))


def _p4_body(a_ref, gm_ref, s2_ref, g2_ref, b2_ref, w2t_ref, bc2_ref,
             z2_ref, s3_ref):
    i = pl.program_id(0)
    h = _gelu(_bn_from(a_ref[...] + gm_ref[...], s2_ref[...], g2_ref[...],
                       b2_ref[...]))
    z = _mm(h, w2t_ref[...]) + bc2_ref[...]
    z2_ref[...] = z
    _acc(i, s3_ref, _sums(z))


def _p5_body(z2_ref, s3_ref, g3_ref, b3_ref, xt_ref, wf1t_ref, bf1_ref,
             x2_ref, u_ref, s4_ref):
    i = pl.program_id(0)
    x2 = _bn_from(z2_ref[...], s3_ref[...], g3_ref[...], b3_ref[...]) \
        + xt_ref[...]
    x2_ref[...] = x2
    u = _mm(x2, wf1t_ref[...]) + bf1_ref[...]
    u_ref[...] = u
    _acc(i, s4_ref, _sums(u))


def _p6_body(u_ref, s4_ref, gf1_ref, bef1_ref, wf2t_ref, bf2_ref,
             v_ref, s5_ref):
    i = pl.program_id(0)
    hu = _gelu(_bn_from(u_ref[...], s4_ref[...], gf1_ref[...], bef1_ref[...]))
    v = _mm(hu, wf2t_ref[...]) + bf2_ref[...]
    v_ref[...] = v
    _acc(i, s5_ref, _sums(v))


def _p7_body(v_ref, s5_ref, gf2_ref, bef2_ref, x2_ref, out_ref):
    out_ref[...] = _bn_from(v_ref[...], s5_ref[...], gf2_ref[...],
                            bef2_ref[...]) + x2_ref[...]


def _row(v):
    return v.reshape(1, -1)


def _chunk_spec(width):
    return pl.BlockSpec((_RC, width), lambda i: (i, 0))


def _full_spec(shape):
    return pl.BlockSpec(shape, lambda i: (0, 0))


def _sum_spec(width):
    return pl.BlockSpec((2, width), lambda i: (0, 0))


def kernel(x, g_fc1_w, g_fc1_b, g_bn1_g, g_bn1_b, g_gc_w, g_gc_b, g_bn2_g,
           g_bn2_b, g_fc2_w, g_fc2_b, g_bn3_g, g_bn3_b, f_fc1_w, f_fc1_b,
           f_bn1_g, f_bn1_b, f_fc2_w, f_fc2_b, f_bn2_g, f_bn2_b):
    xt = x.reshape(BB, CC, NN).transpose(0, 2, 1).reshape(NT, CC)
    wa = (g_gc_w[:, :CC] - g_gc_w[:, CC:]).T
    wb = g_gc_w[:, CC:].T
    f32 = jnp.float32

    z1, s1 = pl.pallas_call(
        _p1_body, grid=(_G,),
        in_specs=[_chunk_spec(CC), _full_spec((CC, CC)), _full_spec((1, CC))],
        out_specs=[_chunk_spec(CC), _sum_spec(CC)],
        out_shape=[jax.ShapeDtypeStruct((NT, CC), f32),
                   jax.ShapeDtypeStruct((2, CC), f32)],
    )(xt, g_fc1_w.T, _row(g_fc1_b))

    feat, a, bmp = pl.pallas_call(
        _p2_body, grid=(_G,),
        in_specs=[_chunk_spec(CC), _sum_spec(CC), _full_spec((1, CC)),
                  _full_spec((1, CC)), _full_spec((CC, HG)),
                  _full_spec((CC, HG)), _full_spec((1, HG))],
        out_specs=[_chunk_spec(CC), _chunk_spec(HG), _chunk_spec(_HGP)],
        out_shape=[jax.ShapeDtypeStruct((NT, CC), f32),
                   jax.ShapeDtypeStruct((NT, HG), f32),
                   jax.ShapeDtypeStruct((NT, _HGP), f32)],
    )(z1, s1, _row(g_bn1_g), _row(g_bn1_b), wa, wb, _row(g_gc_b))

    nn_idx = pl.pallas_call(
        _knn_body, grid=(BB,),
        in_specs=[pl.BlockSpec((1, NN, CC), lambda b: (b, 0, 0))],
        out_specs=pl.BlockSpec((1, NN, KK), lambda b: (b, 0, 0)),
        out_shape=jax.ShapeDtypeStruct((BB, NN, KK), jnp.int32),
    )(feat.reshape(BB, NN, CC))

    gm = _sc_gather_max()(nn_idx.reshape(NT * KK), bmp)

    s2 = pl.pallas_call(
        _p3_body, grid=(_G,),
        in_specs=[_chunk_spec(HG), _chunk_spec(HG)],
        out_specs=_sum_spec(HG),
        out_shape=jax.ShapeDtypeStruct((2, HG), f32),
    )(a, gm)

    z2, s3 = pl.pallas_call(
        _p4_body, grid=(_G,),
        in_specs=[_chunk_spec(HG), _chunk_spec(HG), _sum_spec(HG),
                  _full_spec((1, HG)), _full_spec((1, HG)),
                  _full_spec((HG, CC)), _full_spec((1, CC))],
        out_specs=[_chunk_spec(CC), _sum_spec(CC)],
        out_shape=[jax.ShapeDtypeStruct((NT, CC), f32),
                   jax.ShapeDtypeStruct((2, CC), f32)],
    )(a, gm, s2, _row(g_bn2_g), _row(g_bn2_b), g_fc2_w.T, _row(g_fc2_b))

    x2, u, s4 = pl.pallas_call(
        _p5_body, grid=(_G,),
        in_specs=[_chunk_spec(CC), _sum_spec(CC), _full_spec((1, CC)),
                  _full_spec((1, CC)), _chunk_spec(CC),
                  _full_spec((CC, HF)), _full_spec((1, HF))],
        out_specs=[_chunk_spec(CC), _chunk_spec(HF), _sum_spec(HF)],
        out_shape=[jax.ShapeDtypeStruct((NT, CC), f32),
                   jax.ShapeDtypeStruct((NT, HF), f32),
                   jax.ShapeDtypeStruct((2, HF), f32)],
    )(z2, s3, _row(g_bn3_g), _row(g_bn3_b), xt, f_fc1_w.T, _row(f_fc1_b))

    v, s5 = pl.pallas_call(
        _p6_body, grid=(_G,),
        in_specs=[_chunk_spec(HF), _sum_spec(HF), _full_spec((1, HF)),
                  _full_spec((1, HF)), _full_spec((HF, CC)),
                  _full_spec((1, CC))],
        out_specs=[_chunk_spec(CC), _sum_spec(CC)],
        out_shape=[jax.ShapeDtypeStruct((NT, CC), f32),
                   jax.ShapeDtypeStruct((2, CC), f32)],
    )(u, s4, _row(f_bn1_g), _row(f_bn1_b), f_fc2_w.T, _row(f_fc2_b))

    out = pl.pallas_call(
        _p7_body, grid=(_G,),
        in_specs=[_chunk_spec(CC), _sum_spec(CC), _full_spec((1, CC)),
                  _full_spec((1, CC)), _chunk_spec(CC)],
        out_specs=_chunk_spec(CC),
        out_shape=jax.ShapeDtypeStruct((NT, CC), f32),
    )(v, s5, _row(f_bn2_g), _row(f_bn2_b), x2)

    return out.reshape(BB, NN, CC).transpose(0, 2, 1).reshape(BB, CC, NN, 1)
